# baseline (speedup 1.0000x reference)
import functools

import jax
import jax.numpy as jnp
from jax import lax
from jax.experimental import pallas as pl
from jax.experimental.pallas import tpu as pltpu
from jax.experimental.pallas import tpu_sc as plsc

_B, _C = 16384, 8
_OP_VOCAB, _OP_DIM = 64, 32
_COL_DIM = 16
_STATS_H, _PRED_DIM, _CSTAT_DIM, _OUT_DIM = 16, 8, 8, 64
_BLK = 1024
_GRID = _B // _BLK


@functools.lru_cache(maxsize=None)
def _build_colmean():
    info = plsc.get_sparse_core_info()
    nc, ns = info.num_cores, info.num_subcores
    nw = nc * ns
    idx_w = _B * _C // nw
    row_w = _B // nw

    mesh = plsc.VectorSubcoreMesh(core_axis_name="c", subcore_axis_name="s")

    @functools.partial(
        pl.kernel,
        mesh=mesh,
        out_type=jax.ShapeDtypeStruct((_B, _COL_DIM), jnp.float32),
        compiler_params=pltpu.CompilerParams(use_tc_tiling_on_sc=False),
        scratch_types=[
            pltpu.VMEM((idx_w,), jnp.int32),
            pltpu.VMEM((idx_w, _COL_DIM), jnp.float32),
            pltpu.VMEM((row_w, _COL_DIM), jnp.float32),
            pltpu.SemaphoreType.DMA,
        ],
    )
    def colmean(ids_hbm, table_hbm, out_hbm, idx_v, rows_v, acc_v, sem):
        wid = lax.axis_index("s") * nc + lax.axis_index("c")
        pltpu.sync_copy(ids_hbm.at[pl.ds(wid * idx_w, idx_w)], idx_v)
        pltpu.async_copy(table_hbm.at[idx_v], rows_v, sem).wait()

        def body(b, carry):
            acc = rows_v[b * _C, :]
            for c in range(1, _C):
                acc = acc + rows_v[b * _C + c, :]
            acc_v[b, :] = acc * (1.0 / _C)
            return carry

        lax.fori_loop(0, row_w, body, 0)
        pltpu.sync_copy(acc_v, out_hbm.at[pl.ds(wid * row_w, row_w)])

    return colmean


def _dense_body(opid_ref, stats_ref, pred_ref, cstat_ref, cemb_ref,
                optab_ref, w1t_ref, b1_ref, w2t_ref, b2_ref,
                wct_ref, bc_ref, wot_ref, bo_ref, out_ref):
    f32 = jnp.float32
    opid = lax.broadcast_in_dim(opid_ref[...], (_BLK, _OP_VOCAB), (0,))
    iota = lax.broadcasted_iota(jnp.int32, (_BLK, _OP_VOCAB), 1)
    onehot = (iota == opid).astype(f32)
    op_vec = jnp.dot(onehot, optab_ref[...], preferred_element_type=f32)

    h = jnp.dot(stats_ref[...], w1t_ref[...], preferred_element_type=f32)
    h = jnp.maximum(h + b1_ref[...], 0.0)
    h = jnp.dot(h, w2t_ref[...], preferred_element_type=f32) + b2_ref[...]

    cmean = jnp.sum(cstat_ref[...], axis=1) * (1.0 / _C)
    cs = jnp.dot(cmean, wct_ref[...], preferred_element_type=f32) + bc_ref[...]

    z = jnp.concatenate([op_vec, h, pred_ref[...], cemb_ref[...], cs], axis=-1)
    out_ref[...] = jnp.dot(z, wot_ref[...], preferred_element_type=f32) + bo_ref[...]


def _dense_call(op_idx, stats, pred, cstat, cemb, optab, w1t, b1r, w2t, b2r,
                wct, bcr, wot, bor):
    def row_spec(d):
        return pl.BlockSpec((_BLK, d), lambda i: (i, 0))

    def full_spec(a):
        return pl.BlockSpec(a.shape, lambda i: (0,) * a.ndim)

    return pl.pallas_call(
        _dense_body,
        grid=(_GRID,),
        in_specs=[
            pl.BlockSpec((_BLK,), lambda i: (i,)),
            row_spec(4),
            row_spec(_PRED_DIM),
            pl.BlockSpec((_BLK, _C, 4), lambda i: (i, 0, 0)),
            row_spec(_COL_DIM),
            full_spec(optab),
            full_spec(w1t), full_spec(b1r),
            full_spec(w2t), full_spec(b2r),
            full_spec(wct), full_spec(bcr),
            full_spec(wot), full_spec(bor),
        ],
        out_specs=row_spec(_OUT_DIM),
        out_shape=jax.ShapeDtypeStruct((_B, _OUT_DIM), jnp.float32),
    )(op_idx, stats, pred, cstat, cemb, optab, w1t, b1r, w2t, b2r,
      wct, bcr, wot, bor)


def kernel(op_idx, stats, pred_flags, col_ids, col_stats,
           op_table, col_table, W1, b1, W2, b2, Wc, bc, Wo, bo):
    col_emb = _build_colmean()(col_ids.reshape(-1), col_table)
    return _dense_call(
        op_idx, stats, pred_flags, col_stats, col_emb,
        op_table, W1.T, b1.reshape(1, -1), W2.T, b2.reshape(1, -1),
        Wc.T, bc.reshape(1, -1), Wo.T, bo.reshape(1, -1))


# speedup vs baseline: 1.7638x; 1.7638x over previous
import functools

import jax
import jax.numpy as jnp
from jax import lax
from jax.experimental import pallas as pl
from jax.experimental.pallas import tpu as pltpu
from jax.experimental.pallas import tpu_sc as plsc

_B, _C = 16384, 8
_OP_VOCAB, _OP_DIM = 64, 32
_COL_DIM = 16
_STATS_H, _PRED_DIM, _CSTAT_DIM, _OUT_DIM = 16, 8, 8, 64
_TD = _OP_DIM + _STATS_H + _PRED_DIM + _COL_DIM + _CSTAT_DIM
_BLKT = 2048
_GRIDT = _B // _BLKT


@functools.lru_cache(maxsize=None)
def _build_colmean():
    info = plsc.get_sparse_core_info()
    nc, ns = info.num_cores, info.num_subcores
    nw = nc * ns
    idx_w = _B * _C // nw
    row_w = _B // nw

    mesh = plsc.VectorSubcoreMesh(core_axis_name="c", subcore_axis_name="s")

    @functools.partial(
        pl.kernel,
        mesh=mesh,
        out_type=jax.ShapeDtypeStruct((_B, _COL_DIM), jnp.float32),
        compiler_params=pltpu.CompilerParams(use_tc_tiling_on_sc=False),
        scratch_types=[
            pltpu.VMEM((idx_w,), jnp.int32),
            pltpu.VMEM((idx_w, _COL_DIM), jnp.float32),
            pltpu.VMEM((row_w, _COL_DIM), jnp.float32),
            pltpu.SemaphoreType.DMA,
        ],
    )
    def colmean(ids_hbm, table_hbm, out_hbm, idx_v, rows_v, acc_v, sem):
        wid = lax.axis_index("s") * nc + lax.axis_index("c")
        base = wid * row_w
        for c in range(_C):
            pltpu.sync_copy(ids_hbm.at[pl.ds(c * _B + base, row_w)],
                            idx_v.at[pl.ds(c * row_w, row_w)])
        pltpu.async_copy(table_hbm.at[idx_v], rows_v, sem).wait()

        def body(i, carry):
            acc = rows_v[i, :]
            for c in range(1, _C):
                acc = acc + rows_v[c * row_w + i, :]
            acc_v[i, :] = acc * (1.0 / _C)
            return carry

        lax.fori_loop(0, row_w, body, 0)
        pltpu.sync_copy(acc_v, out_hbm.at[pl.ds(base, row_w)])

    return colmean


def _dense_body(opid_ref, stats_t_ref, pred_t_ref, cstat_t_ref, cemb_ref,
                optab_t_ref, w1_ref, b1_ref, w2_ref, b2_ref,
                wc_ref, bc_ref, wo_ref, bo_ref, out_ref):
    f32 = jnp.float32
    wo = wo_ref[...]

    opid = lax.broadcast_in_dim(opid_ref[...], (_OP_VOCAB, _BLKT), (1,))
    iota = lax.broadcasted_iota(jnp.int32, (_OP_VOCAB, _BLKT), 0)
    onehot = (iota == opid).astype(f32)
    opv_t = jnp.dot(optab_t_ref[...], onehot, preferred_element_type=f32)

    h = jnp.dot(w1_ref[...], stats_t_ref[...], preferred_element_type=f32)
    h = jnp.maximum(h + b1_ref[...], 0.0)
    h = jnp.dot(w2_ref[...], h, preferred_element_type=f32) + b2_ref[...]

    cmean = jnp.sum(cstat_t_ref[...], axis=0) * (1.0 / _C)
    cs = jnp.dot(wc_ref[...], cmean, preferred_element_type=f32) + bc_ref[...]

    cemb_t = jnp.transpose(cemb_ref[...])

    o = jnp.dot(wo[:, 0:_OP_DIM], opv_t, preferred_element_type=f32)
    o = o + jnp.dot(wo[:, _OP_DIM:_OP_DIM + _STATS_H], h,
                    preferred_element_type=f32)
    o = o + jnp.dot(wo[:, 48:48 + _PRED_DIM], pred_t_ref[...],
                    preferred_element_type=f32)
    o = o + jnp.dot(wo[:, 56:56 + _COL_DIM], cemb_t,
                    preferred_element_type=f32)
    o = o + jnp.dot(wo[:, 72:80], cs, preferred_element_type=f32)
    out_ref[...] = o + bo_ref[...]


def _dense_call(op_idx, stats_t, pred_t, cstat_t, cemb,
                optab_t, w1, b1c, w2, b2c, wc, bcc, wo, boc):
    def col_spec(d):
        return pl.BlockSpec((d, _BLKT), lambda i: (0, i))

    def full_spec(a):
        return pl.BlockSpec(a.shape, lambda i: (0,) * a.ndim)

    return pl.pallas_call(
        _dense_body,
        grid=(_GRIDT,),
        in_specs=[
            pl.BlockSpec((_BLKT,), lambda i: (i,)),
            col_spec(4),
            col_spec(_PRED_DIM),
            pl.BlockSpec((_C, 4, _BLKT), lambda i: (0, 0, i)),
            pl.BlockSpec((_BLKT, _COL_DIM), lambda i: (i, 0)),
            full_spec(optab_t),
            full_spec(w1), full_spec(b1c),
            full_spec(w2), full_spec(b2c),
            full_spec(wc), full_spec(bcc),
            full_spec(wo), full_spec(boc),
        ],
        out_specs=col_spec(_OUT_DIM),
        out_shape=jax.ShapeDtypeStruct((_OUT_DIM, _B), jnp.float32),
    )(op_idx, stats_t, pred_t, cstat_t, cemb,
      optab_t, w1, b1c, w2, b2c, wc, bcc, wo, boc)


def kernel(op_idx, stats, pred_flags, col_ids, col_stats,
           op_table, col_table, W1, b1, W2, b2, Wc, bc, Wo, bo):
    ids_cmajor = col_ids.T.reshape(-1)
    col_emb = _build_colmean()(ids_cmajor, col_table)
    out_t = _dense_call(
        op_idx, stats.T, pred_flags.T, col_stats.transpose(1, 2, 0), col_emb,
        op_table.T, W1, b1.reshape(-1, 1), W2, b2.reshape(-1, 1),
        Wc, bc.reshape(-1, 1), Wo, bo.reshape(-1, 1))
    return out_t.T


# speedup vs baseline: 1.7640x; 1.0001x over previous
import functools

import jax
import jax.numpy as jnp
from jax import lax
from jax.experimental import pallas as pl
from jax.experimental.pallas import tpu as pltpu
from jax.experimental.pallas import tpu_sc as plsc

_B, _C = 16384, 8
_OP_VOCAB, _OP_DIM = 64, 32
_COL_DIM = 16
_STATS_H, _PRED_DIM, _CSTAT_DIM, _OUT_DIM = 16, 8, 8, 64
_TD = _OP_DIM + _STATS_H + _PRED_DIM + _COL_DIM + _CSTAT_DIM
_BLKT = 2048
_GRIDT = _B // _BLKT


@functools.lru_cache(maxsize=None)
def _build_colmean():
    info = plsc.get_sparse_core_info()
    nc, ns = info.num_cores, info.num_subcores
    nw = nc * ns
    idx_w = _B * _C // nw
    row_w = _B // nw

    mesh = plsc.VectorSubcoreMesh(core_axis_name="c", subcore_axis_name="s")

    @functools.partial(
        pl.kernel,
        mesh=mesh,
        out_type=jax.ShapeDtypeStruct((_B, _COL_DIM), jnp.float32),
        compiler_params=pltpu.CompilerParams(use_tc_tiling_on_sc=False),
        scratch_types=[
            pltpu.VMEM((idx_w,), jnp.int32),
            pltpu.VMEM((idx_w, _COL_DIM), jnp.float32),
            pltpu.VMEM((row_w, _COL_DIM), jnp.float32),
            pltpu.SemaphoreType.DMA,
        ],
    )
    def colmean(ids_hbm, table_hbm, out_hbm, idx_v, rows_v, acc_v, sem):
        wid = lax.axis_index("s") * nc + lax.axis_index("c")
        base = wid * row_w
        for c in range(_C):
            pltpu.sync_copy(ids_hbm.at[pl.ds(c * _B + base, row_w)],
                            idx_v.at[pl.ds(c * row_w, row_w)])
        pltpu.async_copy(table_hbm.at[idx_v], rows_v, sem).wait()

        def body(i, carry):
            acc = rows_v[i, :]
            for c in range(1, _C):
                acc = acc + rows_v[c * row_w + i, :]
            acc_v[i, :] = acc * (1.0 / _C)
            return carry

        lax.fori_loop(0, row_w, body, 0)
        pltpu.sync_copy(acc_v, out_hbm.at[pl.ds(base, row_w)])

    return colmean


def _dense_body(opid_ref, stats_t_ref, pred_t_ref, cstat_t_ref, cemb_ref,
                optab_t_ref, w1_ref, b1_ref, w2_ref, b2_ref,
                wc_ref, bc_ref, wo_ref, bo_ref, out_ref):
    f32 = jnp.float32
    wo = wo_ref[...]

    opid = lax.broadcast_in_dim(opid_ref[...], (_OP_VOCAB, _BLKT), (1,))
    iota = lax.broadcasted_iota(jnp.int32, (_OP_VOCAB, _BLKT), 0)
    onehot = (iota == opid).astype(f32)
    opv_t = jnp.dot(optab_t_ref[...], onehot, preferred_element_type=f32)

    h = jnp.dot(w1_ref[...], stats_t_ref[...], preferred_element_type=f32)
    h = jnp.maximum(h + b1_ref[...], 0.0)
    h = jnp.dot(w2_ref[...], h, preferred_element_type=f32) + b2_ref[...]

    cmean = jnp.sum(cstat_t_ref[...], axis=0) * (1.0 / _C)
    cs = jnp.dot(wc_ref[...], cmean, preferred_element_type=f32) + bc_ref[...]

    cemb_t = jnp.transpose(cemb_ref[...])

    o = jnp.dot(wo[:, 0:_OP_DIM], opv_t, preferred_element_type=f32)
    o = o + jnp.dot(wo[:, _OP_DIM:_OP_DIM + _STATS_H], h,
                    preferred_element_type=f32)
    o = o + jnp.dot(wo[:, 48:48 + _PRED_DIM], pred_t_ref[...],
                    preferred_element_type=f32)
    o = o + jnp.dot(wo[:, 56:56 + _COL_DIM], cemb_t,
                    preferred_element_type=f32)
    o = o + jnp.dot(wo[:, 72:80], cs, preferred_element_type=f32)
    out_ref[...] = o + bo_ref[...]


def _dense_call(op_idx, stats_t, pred_t, cstat_t, cemb,
                optab_t, w1, b1c, w2, b2c, wc, bcc, wo, boc):
    def col_spec(d):
        return pl.BlockSpec((d, _BLKT), lambda i: (0, i))

    def full_spec(a):
        return pl.BlockSpec(a.shape, lambda i: (0,) * a.ndim)

    return pl.pallas_call(
        _dense_body,
        grid=(_GRIDT,),
        in_specs=[
            pl.BlockSpec((_BLKT,), lambda i: (i,)),
            col_spec(4),
            col_spec(_PRED_DIM),
            pl.BlockSpec((_C, 4, _BLKT), lambda i: (0, 0, i)),
            pl.BlockSpec((_BLKT, _COL_DIM), lambda i: (i, 0)),
            full_spec(optab_t),
            full_spec(w1), full_spec(b1c),
            full_spec(w2), full_spec(b2c),
            full_spec(wc), full_spec(bcc),
            full_spec(wo), full_spec(boc),
        ],
        out_specs=col_spec(_OUT_DIM),
        out_shape=jax.ShapeDtypeStruct((_OUT_DIM, _B), jnp.float32),
    )(op_idx, stats_t, pred_t, cstat_t, cemb,
      optab_t, w1, b1c, w2, b2c, wc, bcc, wo, boc)


def kernel(op_idx, stats, pred_flags, col_ids, col_stats,
           op_table, col_table, W1, b1, W2, b2, Wc, bc, Wo, bo):
    ids_cmajor = col_ids.T.reshape(-1)
    tbl_lin = lax.optimization_barrier(
        col_table.reshape(12500, 128)).reshape(100000, _COL_DIM)
    col_emb = _build_colmean()(ids_cmajor, tbl_lin)
    out_t = _dense_call(
        op_idx, stats.T, pred_flags.T, col_stats.transpose(1, 2, 0), col_emb,
        op_table.T, W1, b1.reshape(-1, 1), W2, b2.reshape(-1, 1),
        Wc, bc.reshape(-1, 1), Wo, bo.reshape(-1, 1))
    return out_t.T
